# Initial kernel scaffold; baseline (speedup 1.0000x reference)
import jax
import jax.numpy as jnp
from jax.experimental import pallas as pl


def kernel(hidden_states, gate_w, w_gate, w_up, w_down):
    raise NotImplementedError("write your pallas kernel here")


# speedup vs baseline: 1.2602x; 1.2602x over previous
import jax
import jax.numpy as jnp
from jax.experimental import pallas as pl

NUM_EXPERTS = 16
TOP_K = 2
HIDDEN = 1024
MOE_FF = 768
TOKENS = 2048

BT = 256


def _moe_dense_kernel(x_ref, gate_ref, wg_ref, wu_ref, wd_ref, out_ref):
    e = pl.program_id(1)
    x = x_ref[...]

    logits = jnp.dot(x, gate_ref[...], preferred_element_type=jnp.float32)
    probs = jax.nn.softmax(logits, axis=-1)
    lane = jax.lax.broadcasted_iota(jnp.int32, probs.shape, 1)
    m1 = jnp.max(probs, axis=-1, keepdims=True)
    i1 = jnp.min(jnp.where(probs == m1, lane, NUM_EXPERTS), axis=-1, keepdims=True)
    masked = jnp.where(lane == i1, -jnp.inf, probs)
    m2 = jnp.max(masked, axis=-1, keepdims=True)
    i2 = jnp.min(jnp.where(masked == m2, lane, NUM_EXPERTS), axis=-1, keepdims=True)
    denom = m1 + m2
    w_e = jnp.where(i1 == e, m1, jnp.where(i2 == e, m2, 0.0)) / denom

    g = jnp.dot(x, wg_ref[0], preferred_element_type=jnp.float32)
    u = jnp.dot(x, wu_ref[0], preferred_element_type=jnp.float32)
    h = (g * jax.nn.sigmoid(g)) * u
    y = jnp.dot(h, wd_ref[0], preferred_element_type=jnp.float32)

    @pl.when(e == 0)
    def _():
        out_ref[...] = jnp.zeros_like(out_ref)

    out_ref[...] += y * w_e


@jax.jit
def kernel(hidden_states, gate_w, w_gate, w_up, w_down):
    grid = (TOKENS // BT, NUM_EXPERTS)
    return pl.pallas_call(
        _moe_dense_kernel,
        grid=grid,
        in_specs=[
            pl.BlockSpec((BT, HIDDEN), lambda t, e: (t, 0)),
            pl.BlockSpec((HIDDEN, NUM_EXPERTS), lambda t, e: (0, 0)),
            pl.BlockSpec((1, HIDDEN, MOE_FF), lambda t, e: (e, 0, 0)),
            pl.BlockSpec((1, HIDDEN, MOE_FF), lambda t, e: (e, 0, 0)),
            pl.BlockSpec((1, MOE_FF, HIDDEN), lambda t, e: (e, 0, 0)),
        ],
        out_specs=pl.BlockSpec((BT, HIDDEN), lambda t, e: (t, 0)),
        out_shape=jax.ShapeDtypeStruct((TOKENS, HIDDEN), jnp.float32),
    )(hidden_states, gate_w, w_gate, w_up, w_down)
